# baseline (speedup 1.0000x reference)
import functools

import jax
import jax.numpy as jnp
from jax import lax
from jax.experimental import pallas as pl
from jax.experimental.pallas import tpu as pltpu
from jax.experimental.pallas import tpu_sc as plsc

_NC = 2
_NS = 16
_NW = _NC * _NS
_CH = 128
_L = 16


def _wid():
    return lax.axis_index("s") * _NC + lax.axis_index("c")


@functools.lru_cache(maxsize=None)
def _build_transpose(V, D):
    assert D == 64
    VP = V - V % _CH
    NCOL = VP // _CH
    TAIL = V - VP
    PR = V // 2
    assert V % 2 == 0 and TAIL % 2 == 0
    base_cols = NCOL // _NW
    extra = NCOL % _NW
    ngrp = (base_cols + 2) // 2

    mesh = plsc.VectorSubcoreMesh(core_axis_name="c", subcore_axis_name="s")

    @functools.partial(
        pl.kernel,
        mesh=mesh,
        out_type=jax.ShapeDtypeStruct((PR, 2 * D), jnp.float32),
        scratch_types=[
            pltpu.VMEM((2, D, _CH + 16), jnp.float32),
            pltpu.VMEM((2, D, _CH), jnp.float32),
            pltpu.VMEM((TAIL, D), jnp.float32),
            pltpu.VMEM((TAIL // 2, 2 * D), jnp.float32),
        ]
        + [pltpu.SemaphoreType.DMA] * 4,
        compiler_params=pltpu.CompilerParams(use_tc_tiling_on_sc=True, needs_layout_passes=False
        ),
    )
    def tr_kernel(xt_hbm, xtail_hbm, xrm_hbm, s_v, o_v, tail_v, otail_v,
                  g0, g1, w0, w1):
        gsems = (g0, g1)
        wsems = (w0, w1)
        wid = _wid()
        start = wid * base_cols + jnp.minimum(wid, extra)
        cnt = base_cols + jnp.where(wid < extra, 1, 0)

        def issue_in(k, b):
            pltpu.make_async_copy(
                xt_hbm.at[:, pl.ds((start + k) * _CH, _CH)],
                s_v.at[b, :, pl.ds(0, _CH)], gsems[b]).start()

        def wait_in(b):
            pltpu.make_async_copy(
                xt_hbm.at[:, pl.ds(0, _CH)],
                s_v.at[b, :, pl.ds(0, _CH)], gsems[b]).wait()

        def issue_out(k, b):
            pltpu.make_async_copy(
                o_v.at[b],
                xrm_hbm.at[pl.ds((start + k) * D, D), :], wsems[b]).start()

        def wait_out(b):
            pltpu.make_async_copy(
                o_v.at[b], xrm_hbm.at[pl.ds(0, D), :], wsems[b]).wait()

        iota = lax.iota(jnp.int32, _L)

        def transpose_block(b):
            @plsc.parallel_loop(0, D, unroll=16)
            def prow(p):
                vals = []
                for h in range(2):
                    colv = jnp.zeros((_L,), jnp.int32) + (2 * p + h)
                    for qi in range(D // _L):
                        rowv = iota + (_L * qi)
                        vals.append(plsc.load_gather(s_v.at[b], [rowv, colv]))
                for h in range(2):
                    for qi in range(D // _L):
                        o_v[b, p, pl.ds(h * D + _L * qi, _L)] = (
                            vals[h * (D // _L) + qi])

        issue_in(0, 0)
        issue_in(1, 1)

        def group(g, carry):
            for b in range(2):
                k = g * 2 + b

                @pl.when(k < cnt)
                def _():
                    wait_in(b)

                    @pl.when(k >= 2)
                    def _w():
                        wait_out(b)

                    transpose_block(b)
                    issue_out(k, b)

                    @pl.when(k + 2 < cnt)
                    def _p():
                        issue_in(k + 2, b)

            return carry

        lax.fori_loop(0, ngrp, group, 0)
        wait_out(0)
        wait_out(1)

        @pl.when(wid == _NW - 1)
        def _tail():
            pltpu.sync_copy(xtail_hbm, tail_v)
            @plsc.parallel_loop(0, TAIL // 2, unroll=2)
            def prow(p):
                vals = []
                for h in range(2):
                    rowv = jnp.zeros((_L,), jnp.int32) + (2 * p + h)
                    for qi in range(D // _L):
                        colv = iota + (_L * qi)
                        vals.append(plsc.load_gather(tail_v, [rowv, colv]))
                for h in range(2):
                    for qi in range(D // _L):
                        otail_v[p, pl.ds(h * D + _L * qi, _L)] = (
                            vals[h * (D // _L) + qi])
            pltpu.sync_copy(otail_v, xrm_hbm.at[pl.ds(VP // 2, TAIL // 2), :])

    return tr_kernel


@functools.lru_cache(maxsize=None)
def _build_gather(B, PR, D):
    bpw = B // _NW
    nch = bpw // _CH
    assert B % (_NW * _CH) == 0

    mesh = plsc.VectorSubcoreMesh(core_axis_name="c", subcore_axis_name="s")

    @functools.partial(
        pl.kernel,
        mesh=mesh,
        out_type=jax.ShapeDtypeStruct((D, B), jnp.float32),
        scratch_types=[
            pltpu.VMEM((bpw,), jnp.int32),
            pltpu.VMEM((bpw,), jnp.int32),
            pltpu.VMEM((_CH,), jnp.int32),
            pltpu.VMEM((2, _CH, 2 * D + 16), jnp.float32),
            pltpu.VMEM((2, D, _CH), jnp.float32),
        ]
        + [pltpu.SemaphoreType.DMA] * 4,
        compiler_params=pltpu.CompilerParams(use_tc_tiling_on_sc=True, needs_layout_passes=False
        ),
    )
    def gather_kernel(xrm_hbm, idx_hbm, out_hbm, idx_v, pair_v, b64_v,
                      g_v, o_v, g0, g1, w0, w1):
        gsems = (g0, g1)
        wsems = (w0, w1)
        wid = _wid()
        base = wid * bpw

        pltpu.sync_copy(idx_hbm.at[pl.ds(base, bpw)], idx_v)

        def mkpair(i, carry):
            iv = idx_v[pl.ds(i * _L, _L)]
            pair_v[pl.ds(i * _L, _L)] = lax.shift_right_logical(iv, 1)
            return carry

        lax.fori_loop(0, bpw // _L, mkpair, 0)

        def issue_gather(k, b):
            pltpu.make_async_copy(
                xrm_hbm.at[pair_v.at[pl.ds(k * _CH, _CH)]],
                g_v.at[b, :, pl.ds(0, 2 * D)], gsems[b]).start()

        def wait_gather(b):
            pltpu.make_async_copy(
                xrm_hbm.at[pl.ds(0, _CH)],
                g_v.at[b, :, pl.ds(0, 2 * D)], gsems[b]).wait()

        def issue_out(k, b):
            pltpu.make_async_copy(
                o_v.at[b],
                out_hbm.at[:, pl.ds(base + k * _CH, _CH)], wsems[b]).start()

        def wait_out(b):
            pltpu.make_async_copy(
                o_v.at[b], out_hbm.at[:, pl.ds(0, _CH)], wsems[b]).wait()

        iota = lax.iota(jnp.int32, _L)

        def extract(k, b):
            for t in range(_CH // _L):
                iv = idx_v[pl.ds(k * _CH + t * _L, _L)]
                b64_v[pl.ds(t * _L, _L)] = lax.shift_left(iv & 1, 6)

            @plsc.parallel_loop(0, D, unroll=8)
            def jrow(j):
                vals = []
                for t in range(_CH // _L):
                    rowv = iota + (t * _L)
                    colv = b64_v[pl.ds(t * _L, _L)] + j
                    vals.append(plsc.load_gather(g_v.at[b], [rowv, colv]))
                for t in range(_CH // _L):
                    o_v[b, j, pl.ds(t * _L, _L)] = vals[t]

        issue_gather(0, 0)
        issue_gather(1, 1)

        def group(g, carry):
            for b in range(2):
                k = g * 2 + b
                wait_gather(b)

                @pl.when(k >= 2)
                def _w():
                    wait_out(b)

                extract(k, b)
                issue_out(k, b)

                @pl.when(k + 2 < nch)
                def _p():
                    issue_gather(k + 2, b)

            return carry

        lax.fori_loop(0, nch // 2, group, 0)
        wait_out(0)
        wait_out(1)

    return gather_kernel


def kernel(x, indices):
    V, D = x.shape
    (B,) = indices.shape
    idx = indices.astype(jnp.int32)
    VP = V - V % _CH

    xt = x.T
    xtail = lax.slice(x, (VP, 0), (V, D))
    x_rm = _build_transpose(V, D)(xt, xtail)
    out_t = _build_gather(B, V // 2, D)(x_rm, idx)
    return out_t.T


# speedup vs baseline: 2.2079x; 2.2079x over previous
import functools

import jax
import jax.numpy as jnp
from jax import lax
from jax.experimental import pallas as pl
from jax.experimental.pallas import tpu as pltpu
from jax.experimental.pallas import tpu_sc as plsc

_NC = 2
_NS = 16
_NW = _NC * _NS
_CH = 128
_NBUF = 4


@functools.lru_cache(maxsize=None)
def _build(B, V, D):
    assert B % (_NW * _CH) == 0, (B, _NW, _CH)
    bpw = B // _NW
    nch = bpw // _CH
    ngrp = nch // _NBUF
    assert nch % _NBUF == 0, (nch, _NBUF)

    mesh = plsc.VectorSubcoreMesh(core_axis_name="c", subcore_axis_name="s")

    @functools.partial(
        pl.kernel,
        mesh=mesh,
        out_type=jax.ShapeDtypeStruct((B, D), jnp.float32),
        scratch_types=[
            pltpu.VMEM((bpw,), jnp.int32),
            pltpu.VMEM((_NBUF, _CH, D), jnp.float32),
        ]
        + [pltpu.SemaphoreType.DMA] * (2 * _NBUF),
        compiler_params=pltpu.CompilerParams(use_tc_tiling_on_sc=False),
        cost_estimate=pl.CostEstimate(
            flops=0,
            transcendentals=0,
            bytes_accessed=2 * B * D * 4 + B * 4,
        ),
    )
    def gather_kernel(x_hbm, idx_hbm, out_hbm, idx_v, rows_v, *sems):
        gsems = sems[:_NBUF]
        wsems = sems[_NBUF:]
        wid = lax.axis_index("s") * _NC + lax.axis_index("c")
        base = wid * bpw

        pltpu.sync_copy(idx_hbm.at[pl.ds(base, bpw)], idx_v)

        def start_gather(j, b):
            pltpu.make_async_copy(
                x_hbm.at[idx_v.at[pl.ds(j * _CH, _CH)]],
                rows_v.at[b],
                gsems[b],
            ).start()

        def wait_gather(b):
            pltpu.make_async_copy(
                x_hbm.at[pl.ds(0, _CH)], rows_v.at[b], gsems[b]
            ).wait()

        def start_write(j, b):
            pltpu.make_async_copy(
                rows_v.at[b],
                out_hbm.at[pl.ds(base + j * _CH, _CH)],
                wsems[b],
            ).start()

        def wait_write(b):
            pltpu.make_async_copy(
                rows_v.at[b], out_hbm.at[pl.ds(0, _CH)], wsems[b]
            ).wait()

        for b in range(_NBUF):
            start_gather(b, b)

        def group(g, carry):
            for b in range(_NBUF):
                wait_gather(b)
                start_write(g * _NBUF + b, b)

            @pl.when(g + 1 < ngrp)
            def _():
                for b in range(_NBUF):
                    wait_write(b)
                    start_gather((g + 1) * _NBUF + b, b)

            return carry

        lax.fori_loop(0, ngrp, group, 0)

        for b in range(_NBUF):
            wait_write(b)

    return gather_kernel


def kernel(x, indices):
    V, D = x.shape
    (B,) = indices.shape
    idx = indices.astype(jnp.int32)
    return _build(B, V, D)(x, idx)
